# baseline (speedup 1.0000x reference)
import math

import jax
import jax.numpy as jnp
from jax.experimental import pallas as pl
from jax.experimental.pallas import tpu as pltpu

_HALF_LOG_2PI = 0.5 * math.log(2.0 * math.pi)


def _mog_block(z_ref, means_ref, logvars_ref, w_ref, out_ref):
    z = z_ref[...]
    mu = means_ref[...]
    lv = logvars_ref[...]
    w = w_ref[...]
    K = mu.shape[0]

    wmax = jnp.max(w, axis=0, keepdims=True)
    logw = (w - wmax) - jnp.log(jnp.sum(jnp.exp(w - wmax), axis=0, keepdims=True))

    A = -0.5 * jnp.exp(-lv)
    Bc = (-2.0 * A) * mu
    C = (logw - _HALF_LOG_2PI - 0.5 * lv) + A * mu * mu

    z2 = z * z
    m = A[0:1, :] * z2 + Bc[0:1, :] * z + C[0:1, :]
    for k in range(1, K):
        x = A[k : k + 1, :] * z2 + Bc[k : k + 1, :] * z + C[k : k + 1, :]
        m = jnp.maximum(m, x)
    s = jnp.zeros(z.shape, jnp.float32)
    for k in range(K):
        x = A[k : k + 1, :] * z2 + Bc[k : k + 1, :] * z + C[k : k + 1, :]
        s = s + jnp.exp(x - m)
    out_ref[...] = -(m + jnp.log(s)).T


def kernel(z, means, logvars, w):
    B, L = z.shape
    K = means.shape[0]
    Bt = 512
    w_b = jnp.broadcast_to(w.reshape(K, 1), (K, L))
    return pl.pallas_call(
        _mog_block,
        grid=(B // Bt,),
        in_specs=[
            pl.BlockSpec((Bt, L), lambda i: (i, 0)),
            pl.BlockSpec((K, L), lambda i: (0, 0)),
            pl.BlockSpec((K, L), lambda i: (0, 0)),
            pl.BlockSpec((K, L), lambda i: (0, 0)),
        ],
        out_specs=pl.BlockSpec((L, Bt), lambda i: (0, i)),
        out_shape=jax.ShapeDtypeStruct((L, B), jnp.float32),
        compiler_params=pltpu.CompilerParams(
            dimension_semantics=("arbitrary",),
        ),
    )(z, means, logvars, w_b)
